# baseline (speedup 1.0000x reference)
import jax
import jax.numpy as jnp
from jax import lax
from jax.experimental import pallas as pl
from jax.experimental.pallas import tpu as pltpu
from jax.experimental.pallas import tpu_sc as plsc

N_VOCAB = 49408
N_EMBED = 768
N_TOKENS = 77
BATCH = 1024

NC = 2
NS = 16
NW = NC * NS
ROWS = BATCH * N_TOKENS
RPW = ROWS // NW
C = 16
NCHUNK = RPW // C
NBUF = 4
POSPAD = 96
PWIN = 16
LANES = 16


def _body(idx_hbm, pose_hbm, table_hbm, out_hbm,
          idx_v, pos_v, b0, b1, b2, b3, g0, g1, g2, g3, w0, w1, w2, w3):
    bufs = (b0, b1, b2, b3)
    gsems = (g0, g1, g2, g3)
    wsems = (w0, w1, w2, w3)

    c = lax.axis_index("c")
    s = lax.axis_index("s")
    wid = s * NC + c
    wbase = wid * RPW

    t_first = wbase // BATCH
    pwin0 = (t_first // 8) * 8

    pltpu.sync_copy(idx_hbm.at[pl.ds(wbase, RPW)], idx_v)
    pltpu.sync_copy(pose_hbm.at[pl.ds(pwin0, PWIN)], pos_v)

    def gdesc(k, bi):
        return pltpu.make_async_copy(
            table_hbm.at[idx_v.at[pl.ds(k * C, C)]], bufs[bi], gsems[bi])

    def wdesc(k, bi):
        return pltpu.make_async_copy(
            bufs[bi], out_hbm.at[pl.ds(wbase + k * C, C)], wsems[bi])

    def add_pos(k, bi):
        buf = bufs[bi]
        tl = (wbase + k * C) // BATCH - pwin0

        nsl = N_EMBED // LANES
        for half in range(2):
            sls = [pl.ds((half * (nsl // 2) + j) * LANES, LANES)
                   for j in range(nsl // 2)]
            pv = [pos_v[tl, sl] for sl in sls]

            def row(i, carry):
                for j, sl in enumerate(sls):
                    buf[i, sl] += pv[j]
                return carry

            lax.fori_loop(0, C, row, 0, unroll=4)

    gdesc(0, 0).start()
    gdesc(1, 1).start()
    for k in (0, 1):
        gdesc(k + 2, k + 2).start()
        gdesc(k, k).wait()
        add_pos(k, k)
        wdesc(k, k).start()

    def group(g, carry):
        for i in range(NBUF):
            k = 2 + g * NBUF + i
            bi = (2 + i) % NBUF
            pi = i

            @pl.when(k < NCHUNK - 2)
            def _():
                wdesc(k - 2, pi).wait()
                gdesc(k + 2, pi).start()

            gdesc(k, bi).wait()
            add_pos(k, bi)
            wdesc(k, bi).start()
        return carry

    lax.fori_loop(0, (NCHUNK - 2) // NBUF, group, 0)

    for k in range(NCHUNK - NBUF, NCHUNK):
        wdesc(k, k % NBUF).wait()


def kernel(x, token_table, position_embedding):
    idx = jnp.swapaxes(x, 0, 1).reshape(-1)
    pos_ext = jnp.concatenate(
        [position_embedding,
         jnp.zeros((POSPAD - N_TOKENS, N_EMBED), jnp.float32)], axis=0)
    mesh = plsc.VectorSubcoreMesh(core_axis_name="c", subcore_axis_name="s")
    k = pl.kernel(
        _body,
        mesh=mesh,
        out_type=jax.ShapeDtypeStruct((ROWS, N_EMBED), jnp.float32),
        scratch_types=(
            [pltpu.VMEM((RPW,), jnp.int32),
             pltpu.VMEM((PWIN, N_EMBED), jnp.float32)]
            + [pltpu.VMEM((C, N_EMBED), jnp.float32)] * NBUF
            + [pltpu.SemaphoreType.DMA] * (2 * NBUF)
        ),
    )
    flat = k(idx, pos_ext, token_table)
    return jnp.swapaxes(flat.reshape(N_TOKENS, BATCH, N_EMBED), 0, 1)


# speedup vs baseline: 1.0689x; 1.0689x over previous
import jax
import jax.numpy as jnp
from jax import lax
from jax.experimental import pallas as pl
from jax.experimental.pallas import tpu as pltpu
from jax.experimental.pallas import tpu_sc as plsc

N_VOCAB = 49408
N_EMBED = 768
N_TOKENS = 77
BATCH = 1024

NC = 2
NS = 16
NW = NC * NS
ROWS = BATCH * N_TOKENS
RPW = ROWS // NW
C = 16
NCHUNK = RPW // C
NBUF = 4
POSPAD = 96
PWIN = 16
LANES = 16


def _body(idx_hbm, pose_hbm, table_hbm, out_hbm,
          idx_v, pos_v, b0, b1, b2, b3, g0, g1, g2, g3, w0, w1, w2, w3):
    bufs = (b0, b1, b2, b3)
    gsems = (g0, g1, g2, g3)
    wsems = (w0, w1, w2, w3)

    c = lax.axis_index("c")
    s = lax.axis_index("s")
    wid = s * NC + c
    wbase = wid * RPW

    t_first = wbase // BATCH
    pwin0 = (t_first // 8) * 8

    pltpu.sync_copy(idx_hbm.at[pl.ds(wbase, RPW)], idx_v)
    pltpu.sync_copy(pose_hbm.at[pl.ds(pwin0, PWIN)], pos_v)

    def gdesc(k, bi):
        return pltpu.make_async_copy(
            table_hbm.at[idx_v.at[pl.ds(k * C, C)]], bufs[bi], gsems[bi])

    def wdesc(k, bi):
        return pltpu.make_async_copy(
            bufs[bi], out_hbm.at[pl.ds(wbase + k * C, C)], wsems[bi])

    def add_pos(k, bi):
        buf = bufs[bi]
        tl = (wbase + k * C) // BATCH - pwin0

        nsl = N_EMBED // LANES
        for half in range(2):
            sls = [pl.ds((half * (nsl // 2) + j) * LANES, LANES)
                   for j in range(nsl // 2)]
            pv = [pos_v[tl, sl] for sl in sls]

            def row(i, carry):
                for j, sl in enumerate(sls):
                    buf[i, sl] += pv[j]
                return carry

            lax.fori_loop(0, C, row, 0)

    gdesc(0, 0).start()
    gdesc(1, 1).start()
    for k in (0, 1):
        gdesc(k + 2, k + 2).start()
        gdesc(k, k).wait()
        add_pos(k, k)
        wdesc(k, k).start()

    def group(g, carry):
        for i in range(NBUF):
            k = 2 + g * NBUF + i
            bi = (2 + i) % NBUF
            pi = i

            @pl.when(k < NCHUNK - 2)
            def _():
                wdesc(k - 2, pi).wait()
                gdesc(k + 2, pi).start()

            gdesc(k, bi).wait()
            add_pos(k, bi)
            wdesc(k, bi).start()
        return carry

    lax.fori_loop(0, (NCHUNK - 2) // NBUF, group, 0)

    for k in range(NCHUNK - NBUF, NCHUNK):
        wdesc(k, k % NBUF).wait()


def kernel(x, token_table, position_embedding):
    idx = jnp.swapaxes(x, 0, 1).reshape(-1)
    pos_ext = jnp.concatenate(
        [position_embedding,
         jnp.zeros((POSPAD - N_TOKENS, N_EMBED), jnp.float32)], axis=0)
    mesh = plsc.VectorSubcoreMesh(core_axis_name="c", subcore_axis_name="s")
    k = pl.kernel(
        _body,
        mesh=mesh,
        out_type=jax.ShapeDtypeStruct((ROWS, N_EMBED), jnp.float32),
        scratch_types=(
            [pltpu.VMEM((RPW,), jnp.int32),
             pltpu.VMEM((PWIN, N_EMBED), jnp.float32)]
            + [pltpu.VMEM((C, N_EMBED), jnp.float32)] * NBUF
            + [pltpu.SemaphoreType.DMA] * (2 * NBUF)
        ),
    )
    flat = k(idx, pos_ext, token_table)
    return jnp.swapaxes(flat.reshape(N_TOKENS, BATCH, N_EMBED), 0, 1)
